# baseline (speedup 1.0000x reference)
import functools

import jax
import jax.numpy as jnp
from jax import lax
from jax.experimental import pallas as pl
from jax.experimental.pallas import tpu as pltpu
from jax.experimental.pallas import tpu_sc as plsc

NUM_CLS = 1000
B = 16384
LANES = 16
NUM_CORES = 2
NUM_SUBCORES = 16
NW = NUM_CORES * NUM_SUBCORES
ROWS_PER_W = B // NW
CHUNK_ROWS = 64
CHUNK_WORDS = CHUNK_ROWS * NUM_CLS
NCHUNK = ROWS_PER_W // CHUNK_ROWS
GROUPS = CHUNK_ROWS // LANES


def _sc_body(x_hbm, out_hbm, idx_v, buf0, buf1, sem0, sem1):
    cid = lax.axis_index("c")
    sid = lax.axis_index("s")
    wid = sid * NUM_CORES + cid
    rbase = wid * ROWS_PER_W

    pltpu.sync_copy(x_hbm.at[pl.ds(rbase, ROWS_PER_W)], idx_v)

    zeros16 = jnp.zeros((LANES,), jnp.float32)
    ones16 = jnp.ones((LANES,), jnp.float32)
    lane = lax.iota(jnp.int32, LANES)

    def zero_loop(i, _):
        buf0[pl.ds(i * LANES, LANES)] = zeros16
        buf1[pl.ds(i * LANES, LANES)] = zeros16
        return ()

    lax.fori_loop(0, CHUNK_WORDS // LANES, zero_loop, ())

    def scatter_chunk(c, buf, vals):
        for g in range(GROUPS):
            xv = idx_v[pl.ds(c * CHUNK_ROWS + g * LANES, LANES)]
            pos = (lane + g * LANES) * NUM_CLS + xv
            plsc.store_scatter(buf, [pos], vals)

    bufs = (buf0, buf1)
    sems = (sem0, sem1)
    pending = [None, None]
    flat_base = rbase * NUM_CLS
    for c in range(NCHUNK):
        b = c % 2
        buf = bufs[b]
        if pending[b] is not None:
            pending[b].wait()
            scatter_chunk(c - 2, buf, zeros16)
        scatter_chunk(c, buf, ones16)
        cp = pltpu.make_async_copy(
            buf, out_hbm.at[pl.ds(flat_base + c * CHUNK_WORDS, CHUNK_WORDS)], sems[b]
        )
        cp.start()
        pending[b] = cp
    pending[0].wait()
    pending[1].wait()


_mesh = plsc.VectorSubcoreMesh(core_axis_name="c", subcore_axis_name="s")

_sc_call = functools.partial(
    pl.kernel,
    out_type=jax.ShapeDtypeStruct((B * NUM_CLS,), jnp.float32),
    mesh=_mesh,
    compiler_params=pltpu.CompilerParams(
        use_tc_tiling_on_sc=False, needs_layout_passes=False
    ),
    scratch_types=[
        pltpu.VMEM((ROWS_PER_W,), jnp.int32),
        pltpu.VMEM((CHUNK_WORDS,), jnp.float32),
        pltpu.VMEM((CHUNK_WORDS,), jnp.float32),
        pltpu.SemaphoreType.DMA,
        pltpu.SemaphoreType.DMA,
    ],
)(_sc_body)


def kernel(x):
    return _sc_call(x).reshape(B, NUM_CLS)


# speedup vs baseline: 1.0023x; 1.0023x over previous
import functools

import jax
import jax.numpy as jnp
from jax import lax
from jax.experimental import pallas as pl
from jax.experimental.pallas import tpu as pltpu
from jax.experimental.pallas import tpu_sc as plsc

NUM_CLS = 1000
B = 16384
LANES = 16
NUM_CORES = 2
NUM_SUBCORES = 16
NW = NUM_CORES * NUM_SUBCORES
ROWS_PER_W = B // NW
CHUNK_ROWS = 64
NCHUNK = ROWS_PER_W // CHUNK_ROWS
GROUPS = CHUNK_ROWS // LANES
COL_GROUPS = -(-NUM_CLS // LANES)


def _sc_body(x_hbm, out_hbm, idx_v, buf0, buf1, sem0, sem1):
    cid = lax.axis_index("c")
    sid = lax.axis_index("s")
    wid = sid * NUM_CORES + cid
    rbase = wid * ROWS_PER_W

    pltpu.sync_copy(x_hbm.at[pl.ds(rbase, ROWS_PER_W)], idx_v)

    zeros16 = jnp.zeros((LANES,), jnp.float32)
    ones16 = jnp.ones((LANES,), jnp.float32)
    lane = lax.iota(jnp.int32, LANES)

    def zero_row(r, _):
        def zero_grp(g, _):
            off = jnp.minimum(g * LANES, NUM_CLS - LANES)
            buf0[r, pl.ds(off, LANES)] = zeros16
            buf1[r, pl.ds(off, LANES)] = zeros16
            return ()

        return lax.fori_loop(0, COL_GROUPS, zero_grp, ())

    lax.fori_loop(0, CHUNK_ROWS, zero_row, ())

    def scatter_chunk(c, buf, vals):
        for g in range(GROUPS):
            xv = idx_v[pl.ds(c * CHUNK_ROWS + g * LANES, LANES)]
            rows = lane + g * LANES
            plsc.store_scatter(buf, [rows, xv], vals)

    bufs = (buf0, buf1)
    sems = (sem0, sem1)
    pending = [None, None]
    for c in range(NCHUNK):
        b = c % 2
        buf = bufs[b]
        if pending[b] is not None:
            pending[b].wait()
            scatter_chunk(c - 2, buf, zeros16)
        scatter_chunk(c, buf, ones16)
        cp = pltpu.make_async_copy(
            buf, out_hbm.at[pl.ds(rbase + c * CHUNK_ROWS, CHUNK_ROWS)], sems[b]
        )
        cp.start()
        pending[b] = cp
    pending[0].wait()
    pending[1].wait()


_mesh = plsc.VectorSubcoreMesh(core_axis_name="c", subcore_axis_name="s")

kernel = functools.partial(
    pl.kernel,
    out_type=jax.ShapeDtypeStruct((B, NUM_CLS), jnp.float32),
    mesh=_mesh,
    compiler_params=pltpu.CompilerParams(
        use_tc_tiling_on_sc=False, needs_layout_passes=False
    ),
    scratch_types=[
        pltpu.VMEM((ROWS_PER_W,), jnp.int32),
        pltpu.VMEM((CHUNK_ROWS, NUM_CLS), jnp.float32),
        pltpu.VMEM((CHUNK_ROWS, NUM_CLS), jnp.float32),
        pltpu.SemaphoreType.DMA,
        pltpu.SemaphoreType.DMA,
    ],
)(_sc_body)


# speedup vs baseline: 1.6991x; 1.6952x over previous
import functools

import jax
import jax.numpy as jnp
from jax import lax
from jax.experimental import pallas as pl
from jax.experimental.pallas import tpu as pltpu
from jax.experimental.pallas import tpu_sc as plsc

NUM_CLS = 1000
B = 16384
LANES = 16
NUM_CORES = 2
NUM_SUBCORES = 16
NW = NUM_CORES * NUM_SUBCORES
ROWS_PER_W = B // NW
CHUNK_ROWS = 32
NCHUNK = ROWS_PER_W // CHUNK_ROWS
GROUPS = CHUNK_ROWS // LANES
COL_GROUPS = -(-NUM_CLS // LANES)


def _sc_body(x_hbm, out_hbm, idx_v, buf0, buf1, sem0, sem1):
    cid = lax.axis_index("c")
    sid = lax.axis_index("s")
    wid = sid * NUM_CORES + cid
    rbase = wid * ROWS_PER_W

    pltpu.sync_copy(x_hbm.at[pl.ds(rbase, ROWS_PER_W)], idx_v)

    zeros16 = jnp.zeros((LANES,), jnp.float32)
    ones16 = jnp.ones((LANES,), jnp.float32)
    lane = lax.iota(jnp.int32, LANES)

    def zero_row(r, _):
        def zero_grp(g, _):
            off = jnp.minimum(g * LANES, NUM_CLS - LANES)
            buf0[r, pl.ds(off, LANES)] = zeros16
            buf1[r, pl.ds(off, LANES)] = zeros16
            return ()

        return lax.fori_loop(0, COL_GROUPS, zero_grp, ())

    lax.fori_loop(0, CHUNK_ROWS, zero_row, ())

    def scatter_chunk(c, buf, vals):
        for g in range(GROUPS):
            xv = idx_v[pl.ds(c * CHUNK_ROWS + g * LANES, LANES)]
            rows = lane + g * LANES
            plsc.store_scatter(buf, [rows, xv], vals)

    bufs = (buf0, buf1)
    sems = (sem0, sem1)
    pending = [None, None]
    for c in range(NCHUNK):
        b = c % 2
        buf = bufs[b]
        if pending[b] is not None:
            pending[b].wait()
            scatter_chunk(c - 2, buf, zeros16)
        scatter_chunk(c, buf, ones16)
        cp = pltpu.make_async_copy(
            buf, out_hbm.at[pl.ds(rbase + c * CHUNK_ROWS, CHUNK_ROWS)], sems[b]
        )
        cp.start()
        pending[b] = cp
    pending[0].wait()
    pending[1].wait()


_mesh = plsc.VectorSubcoreMesh(core_axis_name="c", subcore_axis_name="s")

kernel = functools.partial(
    pl.kernel,
    out_type=jax.ShapeDtypeStruct((B, NUM_CLS), jnp.float32),
    mesh=_mesh,
    compiler_params=pltpu.CompilerParams(
        use_tc_tiling_on_sc=True, needs_layout_passes=False
    ),
    scratch_types=[
        pltpu.VMEM((ROWS_PER_W,), jnp.int32),
        pltpu.VMEM((CHUNK_ROWS, NUM_CLS), jnp.float32),
        pltpu.VMEM((CHUNK_ROWS, NUM_CLS), jnp.float32),
        pltpu.SemaphoreType.DMA,
        pltpu.SemaphoreType.DMA,
    ],
)(_sc_body)


# speedup vs baseline: 3.7044x; 2.1802x over previous
import functools

import jax
import jax.numpy as jnp
from jax import lax
from jax.experimental import pallas as pl
from jax.experimental.pallas import tpu as pltpu
from jax.experimental.pallas import tpu_sc as plsc

NUM_CLS = 1000
B = 16384
LANES = 16
NUM_CORES = 2
NUM_SUBCORES = 16
NW = NUM_CORES * NUM_SUBCORES
COLS_PER_W = B // NW
CLS_CHUNK = 40
NCHUNK = NUM_CLS // CLS_CHUNK
COL_GROUPS = COLS_PER_W // LANES


def _sc_body(x_hbm, out_hbm, idx_v, buf0, buf1, sem0, sem1):
    cid = lax.axis_index("c")
    sid = lax.axis_index("s")
    wid = sid * NUM_CORES + cid
    cbase = wid * COLS_PER_W

    pltpu.sync_copy(x_hbm.at[pl.ds(cbase, COLS_PER_W)], idx_v)

    zeros16 = jnp.zeros((LANES,), jnp.float32)
    ones16 = jnp.ones((LANES,), jnp.float32)
    lane = lax.iota(jnp.int32, LANES)

    def zero_row(r, _):
        def zero_grp(g, _):
            buf0[r, pl.ds(g * LANES, LANES)] = zeros16
            buf1[r, pl.ds(g * LANES, LANES)] = zeros16
            return ()

        return lax.fori_loop(0, COL_GROUPS, zero_grp, ())

    lax.fori_loop(0, CLS_CHUNK, zero_row, ())

    def scan_chunk(k, buf, vals):
        lo = k * CLS_CHUNK

        def grp(g, _):
            xv = idx_v[pl.ds(g * LANES, LANES)]
            m = (xv >= lo) & (xv < lo + CLS_CHUNK)
            rows = jnp.where(m, xv - lo, 0)
            cols = lane + g * LANES
            plsc.store_scatter(buf, [rows, cols], vals, mask=m)
            return ()

        lax.fori_loop(0, COL_GROUPS, grp, ())

    bufs = (buf0, buf1)
    sems = (sem0, sem1)
    pending = [None, None]
    for k in range(NCHUNK):
        b = k % 2
        buf = bufs[b]
        if pending[b] is not None:
            pending[b].wait()
            scan_chunk(k - 2, buf, zeros16)
        scan_chunk(k, buf, ones16)
        cp = pltpu.make_async_copy(
            buf,
            out_hbm.at[pl.ds(k * CLS_CHUNK, CLS_CHUNK), pl.ds(cbase, COLS_PER_W)],
            sems[b],
        )
        cp.start()
        pending[b] = cp
    pending[0].wait()
    pending[1].wait()


_mesh = plsc.VectorSubcoreMesh(core_axis_name="c", subcore_axis_name="s")

_sc_call = functools.partial(
    pl.kernel,
    out_type=jax.ShapeDtypeStruct((NUM_CLS, B), jnp.float32),
    mesh=_mesh,
    compiler_params=pltpu.CompilerParams(
        use_tc_tiling_on_sc=True, needs_layout_passes=False
    ),
    scratch_types=[
        pltpu.VMEM((COLS_PER_W,), jnp.int32),
        pltpu.VMEM((CLS_CHUNK, COLS_PER_W), jnp.float32),
        pltpu.VMEM((CLS_CHUNK, COLS_PER_W), jnp.float32),
        pltpu.SemaphoreType.DMA,
        pltpu.SemaphoreType.DMA,
    ],
)(_sc_body)


def kernel(x):
    return _sc_call(x).T


# speedup vs baseline: 4.0913x; 1.1044x over previous
import functools

import jax
import jax.numpy as jnp
from jax import lax
from jax.experimental import pallas as pl
from jax.experimental.pallas import tpu as pltpu
from jax.experimental.pallas import tpu_sc as plsc

NUM_CLS = 1000
B = 16384
LANES = 16
NUM_CORES = 2
NUM_SUBCORES = 16
NW = NUM_CORES * NUM_SUBCORES
COLS_PER_W = B // NW
CLS_CHUNK = 120
COL_GROUPS = COLS_PER_W // LANES

_BANDS = [(k * CLS_CHUNK, CLS_CHUNK) for k in range(NUM_CLS // CLS_CHUNK)]
_BANDS.append((NUM_CLS - NUM_CLS % CLS_CHUNK, NUM_CLS % CLS_CHUNK))


def _sc_body(x_hbm, out_hbm, idx_v, buf0, buf1, sem0, sem1):
    cid = lax.axis_index("c")
    sid = lax.axis_index("s")
    wid = sid * NUM_CORES + cid
    cbase = wid * COLS_PER_W

    pltpu.sync_copy(x_hbm.at[pl.ds(cbase, COLS_PER_W)], idx_v)

    zeros16 = jnp.zeros((LANES,), jnp.float32)
    ones16 = jnp.ones((LANES,), jnp.float32)
    lane = lax.iota(jnp.int32, LANES)

    def zero_buf(buf):
        def zero_row(r, _):
            for g in range(COL_GROUPS):
                buf[r, pl.ds(g * LANES, LANES)] = zeros16
            return ()

        lax.fori_loop(0, CLS_CHUNK, zero_row, ())

    def scan_band(lo, hi, buf, vals):
        def grp(g, _):
            xv = idx_v[pl.ds(g * LANES, LANES)]
            m = (xv >= lo) & (xv < hi)
            rows = jnp.where(m, xv - lo, 0)
            cols = lane + g * LANES
            plsc.store_scatter(buf, [rows, cols], vals, mask=m)
            return ()

        lax.fori_loop(0, COL_GROUPS, grp, ())

    bufs = (buf0, buf1)
    sems = (sem0, sem1)
    pending = [None, None]
    for k, (lo, rows) in enumerate(_BANDS):
        b = k % 2
        buf = bufs[b]
        if k < 2:
            zero_buf(buf)
        else:
            pending[b].wait()
            plo, prows = _BANDS[k - 2]
            scan_band(plo, plo + prows, buf, zeros16)
        scan_band(lo, lo + rows, buf, ones16)
        cp = pltpu.make_async_copy(
            buf.at[pl.ds(0, rows)],
            out_hbm.at[pl.ds(lo, rows), pl.ds(cbase, COLS_PER_W)],
            sems[b],
        )
        cp.start()
        pending[b] = cp
    pending[0].wait()
    pending[1].wait()


_mesh = plsc.VectorSubcoreMesh(core_axis_name="c", subcore_axis_name="s")

_sc_call = functools.partial(
    pl.kernel,
    out_type=jax.ShapeDtypeStruct((NUM_CLS, B), jnp.float32),
    mesh=_mesh,
    compiler_params=pltpu.CompilerParams(
        use_tc_tiling_on_sc=True, needs_layout_passes=False
    ),
    scratch_types=[
        pltpu.VMEM((COLS_PER_W,), jnp.int32),
        pltpu.VMEM((CLS_CHUNK, COLS_PER_W), jnp.float32),
        pltpu.VMEM((CLS_CHUNK, COLS_PER_W), jnp.float32),
        pltpu.SemaphoreType.DMA,
        pltpu.SemaphoreType.DMA,
    ],
)(_sc_body)


def kernel(x):
    return _sc_call(x).T
